# baseline (speedup 1.0000x reference)
import functools

import jax
import jax.numpy as jnp
from jax import lax
from jax.experimental import pallas as pl
from jax.experimental.pallas import tpu as pltpu
from jax.experimental.pallas import tpu_sc as plsc

_NC = 2
_NS = 16
_NW = _NC * _NS


def _make_router_body(B, E, T, G, BLK):
    NB = B // BLK

    def body(x_ref, wr_ref, br_ref, inv_ref, sc_ref, te_ref):
        logits = jnp.dot(x_ref[...], wr_ref[...], preferred_element_type=jnp.float32)
        logits = logits + br_ref[...][None, :]
        m = jnp.max(logits, axis=1, keepdims=True)
        ex = jnp.exp(logits - m)
        probs = ex / jnp.sum(ex, axis=1, keepdims=True)
        idx = jnp.argmax(probs, axis=1).astype(jnp.int32)
        sc_ref[...] = jnp.max(probs, axis=1)

        col = lax.broadcasted_iota(jnp.int32, (B, E), 1)
        oh = (col == idx[:, None]).astype(jnp.float32)
        tril = (
            lax.broadcasted_iota(jnp.int32, (BLK, BLK), 0)
            >= lax.broadcasted_iota(jnp.int32, (BLK, BLK), 1)
        ).astype(jnp.float32)
        carry = jnp.zeros((1, E), jnp.float32)
        rws = []
        for b in range(NB):
            ohb = oh[b * BLK:(b + 1) * BLK, :]
            cs = jnp.dot(tril, ohb, preferred_element_type=jnp.float32) + carry
            rws.append(jnp.sum(cs * ohb, axis=1) - 1.0)
            carry = cs[BLK - 1:BLK, :]
        counts = carry
        rw = jnp.concatenate(rws)

        tpe = jnp.floor((counts + (T - 1)) * (1.0 / T))
        le = (
            lax.broadcasted_iota(jnp.int32, (E, E), 0)
            <= lax.broadcasted_iota(jnp.int32, (E, E), 1)
        ).astype(jnp.float32)
        toff = jnp.dot(tpe, le, preferred_element_type=jnp.float32)
        pad_start = (toff - tpe) * T
        ps_tok = jnp.sum(oh * pad_start, axis=1)
        inv_ref[...] = (ps_tok + rw).astype(jnp.int32)

        gb = lax.broadcasted_iota(jnp.int32, (G, E), 0).astype(jnp.float32)
        te = jnp.sum((gb >= toff).astype(jnp.float32), axis=1)
        te_ref[...] = jnp.minimum(te, E - 1).astype(jnp.int32)

    return body


def _ffn_body(te_ref, x_ref, w1_ref, b1_ref, w2_ref, b2_ref, s_ref, o_ref):
    del te_ref
    xb = x_ref[...]
    h = jnp.dot(xb, w1_ref[0], preferred_element_type=jnp.float32) + b1_ref[0]
    h = jnp.maximum(h, 0.0)
    y = jnp.dot(h, w2_ref[0], preferred_element_type=jnp.float32) + b2_ref[0]
    o_ref[...] = y * s_ref[0, 0][:, None]


def _sc_mesh():
    return plsc.VectorSubcoreMesh(
        core_axis_name="c",
        subcore_axis_name="s",
        num_cores=_NC,
        num_subcores=_NS,
    )


def _make_row_gather(n_rows, d, chunk, name):
    per_w = n_rows // _NW
    assert per_w % chunk == 0
    n_ch = per_w // chunk

    def body(table_hbm, idx_hbm, out_hbm, idx_v, rows_v, sem):
        wid = lax.axis_index("s") * _NC + lax.axis_index("c")
        base = wid * per_w
        for c in range(n_ch):
            off = base + c * chunk
            pltpu.sync_copy(idx_hbm.at[pl.ds(off, chunk)], idx_v)
            pltpu.async_copy(table_hbm.at[idx_v], rows_v, sem).wait()
            pltpu.sync_copy(rows_v, out_hbm.at[pl.ds(off, chunk)])

    body.__name__ = name
    return functools.partial(
        pl.kernel,
        mesh=_sc_mesh(),
        out_type=jax.ShapeDtypeStruct((n_rows, d), jnp.float32),
        scratch_types=[
            pltpu.VMEM((chunk,), jnp.int32),
            pltpu.VMEM((chunk, d), jnp.float32),
            pltpu.SemaphoreType.DMA,
        ],
    )(body)


def _make_row_scatter(n_src, d, n_dst, name):
    per_w = n_src // _NW
    assert n_src % _NW == 0

    def body(src_hbm, idx_hbm, out_hbm, idx_v, rows_v, sem):
        wid = lax.axis_index("s") * _NC + lax.axis_index("c")
        base = wid * per_w
        pltpu.sync_copy(idx_hbm.at[wid], idx_v)
        pltpu.sync_copy(src_hbm.at[pl.ds(base, per_w)], rows_v)
        pltpu.async_copy(rows_v, out_hbm.at[idx_v], sem).wait()

    body.__name__ = name
    return functools.partial(
        pl.kernel,
        mesh=_sc_mesh(),
        out_type=jax.ShapeDtypeStruct((n_dst, d), jnp.float32),
        scratch_types=[
            pltpu.VMEM((per_w,), jnp.int32),
            pltpu.VMEM((per_w, d), jnp.float32),
            pltpu.SemaphoreType.DMA,
        ],
    )(body)


def kernel(x, Wr, br, W1, b1, W2, b2):
    B, D = x.shape
    E = Wr.shape[1]
    H = W1.shape[2]
    O = W2.shape[2]
    T = 128
    G = B // T + E
    PB = G * T

    inv_perm, scores, tile_expert = pl.pallas_call(
        _make_router_body(B, E, T, G, 256),
        out_shape=[
            jax.ShapeDtypeStruct((B,), jnp.int32),
            jax.ShapeDtypeStruct((B,), jnp.float32),
            jax.ShapeDtypeStruct((G,), jnp.int32),
        ],
    )(x, Wr, br)
    scores_pad = (
        jnp.zeros((PB,), jnp.float32).at[inv_perm].set(scores).reshape(G, 1, T)
    )

    scatter_in = _make_row_scatter(B, D, PB, "moe_stage_tokens")
    xs = scatter_in(x, inv_perm.reshape(_NW, B // _NW))

    b1r = b1.reshape(E, 1, H)
    b2r = b2.reshape(E, 1, O)
    grid_spec = pltpu.PrefetchScalarGridSpec(
        num_scalar_prefetch=1,
        grid=(G,),
        in_specs=[
            pl.BlockSpec((T, D), lambda g, te: (g, 0)),
            pl.BlockSpec((1, D, H), lambda g, te: (te[g], 0, 0)),
            pl.BlockSpec((1, 1, H), lambda g, te: (te[g], 0, 0)),
            pl.BlockSpec((1, H, O), lambda g, te: (te[g], 0, 0)),
            pl.BlockSpec((1, 1, O), lambda g, te: (te[g], 0, 0)),
            pl.BlockSpec((1, 1, T), lambda g, te: (g, 0, 0)),
        ],
        out_specs=pl.BlockSpec((T, O), lambda g, te: (g, 0)),
    )
    ys = pl.pallas_call(
        _ffn_body,
        grid_spec=grid_spec,
        out_shape=jax.ShapeDtypeStruct((PB, O), jnp.float32),
    )(tile_expert, xs, W1, b1r, W2, b2r, scores_pad)

    gather_out = _make_row_gather(B, O, 64, "moe_unpermute")
    out = gather_out(ys, inv_perm)
    return out
